# baseline (device time: 146380 ns/iter reference)
import jax
import jax.numpy as jnp
from jax import lax
from jax.experimental import pallas as pl
from jax.experimental.pallas import tpu as pltpu

N_Z = 4
N_RING = 8
M = 2048
D = 2048
PIECE = M // N_RING
HALF = PIECE // 2
N_SUB = 8
QTR = PIECE // N_SUB
N_HOP = 4


def _ring_coords(p):
    px = (p >= 4).astype(jnp.int32)
    py = jnp.where(p < 4, p, 7 - p)
    return px, py


def kernel(partial, resid, gamma):
    x = partial.reshape(M, D)
    g = gamma.reshape(1, D)

    def body(x_hbm, resid_hbm, g_ref, out_ref,
             acc_ref, a_comm, b_comm, c_comm, stage_ref,
             a_send, a_recv, b_send, b_recv, c_send, c_recv,
             b_credit, c_credit, local_sem, stage_sems):
        my_x = lax.axis_index("x")
        my_y = lax.axis_index("y")
        my_z = lax.axis_index("z")

        pos = jnp.where(my_x == 0, my_y, 7 - my_y)
        right_x, right_y = _ring_coords((pos + 1) % N_RING)
        left_x, left_y = _ring_coords((pos + 7) % N_RING)

        my_off = pos * PIECE

        is_edge = jnp.logical_or(my_z == 0, my_z == 3)
        partner = my_z ^ 1
        other_mid = 3 - my_z

        bar = pltpu.get_barrier_semaphore()
        for did in (
            (my_x, my_y, partner),
            (left_x, left_y, my_z),
            (right_x, right_y, my_z),
        ):
            pl.semaphore_signal(
                bar, inc=1, device_id=did,
                device_id_type=pl.DeviceIdType.MESH,
            )

        @pl.when(jnp.logical_not(is_edge))
        def _():
            pl.semaphore_signal(
                bar, inc=1, device_id=(my_x, my_y, other_mid),
                device_id_type=pl.DeviceIdType.MESH,
            )

        pl.semaphore_wait(bar, 3)

        @pl.when(jnp.logical_not(is_edge))
        def _():
            pl.semaphore_wait(bar, 1)

        cp = pltpu.make_async_copy(
            x_hbm.at[pl.ds(my_off, PIECE), :], acc_ref, local_sem
        )
        cp.start()

        pf_ctr = [0]

        def prefetch_piece(idx):
            slot = pf_ctr[0] % 4
            pf_ctr[0] += 1
            dma = pltpu.make_async_copy(
                resid_hbm.at[pl.ds(idx * PIECE, PIECE), :],
                stage_ref.at[slot],
                stage_sems.at[slot],
            )
            dma.start()
            return (dma, slot)

        pf_m1 = prefetch_piece((pos - 1) % N_RING)
        pf_p1 = prefetch_piece((pos + 1) % N_RING)
        pf_own = prefetch_piece(pos)

        cp.wait()

        def a_rdma(sub, slot, src, dst_z):
            return pltpu.make_async_remote_copy(
                src_ref=src,
                dst_ref=a_comm.at[sub, slot],
                send_sem=a_send.at[sub, slot],
                recv_sem=a_recv.at[sub, slot],
                device_id=(my_x, my_y, dst_z),
                device_id_type=pl.DeviceIdType.MESH,
            )

        def qtr(sub, ref=None, base=0):
            r = acc_ref if ref is None else ref
            return r.at[pl.ds(base + sub * QTR, QTR), :]

        def half(sub, ref=None, base=0):
            r = acc_ref if ref is None else ref
            return r.at[pl.ds(base + sub * HALF, HALF), :]

        def mk_fwd(dirn, h, j):
            comm, ss, rs = (
                (b_comm, b_send, b_recv) if dirn == "cw"
                else (c_comm, c_send, c_recv)
            )
            dev = (
                (right_x, right_y, my_z) if dirn == "cw"
                else (left_x, left_y, my_z)
            )
            if h == 0:
                src = half(j, out_ref, my_off)
            else:
                src = comm.at[(h - 1) % 2, pl.ds(j * HALF, HALF), :]
            return pltpu.make_async_remote_copy(
                src_ref=src,
                dst_ref=comm.at[h % 2, pl.ds(j * HALF, HALF), :],
                send_sem=ss.at[h % 2, j],
                recv_sem=rs.at[h % 2, j],
                device_id=dev,
                device_id_type=pl.DeviceIdType.MESH,
            )

        cw0 = [mk_fwd("cw", 0, j) for j in range(2)]
        ccw0 = [mk_fwd("ccw", 0, j) for j in range(2)]

        @pl.when(is_edge)
        def _():
            rd = [a_rdma(s, 0, qtr(s), partner) for s in range(N_SUB)]
            for s in range(N_SUB):
                rd[s].start()
            for s in range(N_SUB):
                fin = a_rdma(s, 1, qtr(s), partner)
                fin.wait_recv()
                out_ref[pl.ds(my_off + s * QTR, QTR), :] = a_comm[s, 1]
                if s == N_SUB // 2 - 1:
                    cw0[0].start()
                    ccw0[0].start()
                elif s == N_SUB - 1:
                    cw0[1].start()
                    ccw0[1].start()
            for s in range(N_SUB):
                rd[s].wait_send()

        @pl.when(jnp.logical_not(is_edge))
        def _():
            rcv = [a_rdma(s, 0, qtr(s), partner) for s in range(N_SUB)]
            ex = [None] * N_SUB
            for s in range(N_SUB):
                rcv[s].wait_recv()
                acc_ref[pl.ds(s * QTR, QTR), :] = (
                    acc_ref[pl.ds(s * QTR, QTR), :] + a_comm[s, 0]
                )
                ex[s] = a_rdma(s, 1, qtr(s), other_mid)
                ex[s].start()
            fin = [None] * N_SUB
            for s in range(N_SUB):
                ex[s].wait()
                out_ref[pl.ds(my_off + s * QTR, QTR), :] = (
                    acc_ref[pl.ds(s * QTR, QTR), :] + a_comm[s, 1]
                )
                fin[s] = a_rdma(
                    s, 1, qtr(s, out_ref, my_off), partner
                )
                fin[s].start()
                if s == N_SUB // 2 - 1:
                    cw0[0].start()
                    ccw0[0].start()
                elif s == N_SUB - 1:
                    cw0[1].start()
                    ccw0[1].start()
            for s in range(N_SUB):
                fin[s].wait_send()

        def ln_from(pf, row0, src_block, nrows=PIECE, sub_off=0, wait=True):
            dma, slot = pf
            if wait:
                dma.wait()
            if nrows == PIECE:
                st = stage_ref[slot]
            elif sub_off == 0:
                st = stage_ref[slot, :HALF, :]
            else:
                st = stage_ref[slot, HALF:, :]
            y = src_block + st
            ms = jnp.mean(y * y, axis=1, keepdims=True)
            inv = lax.rsqrt(ms + 1e-6)
            out_ref[pl.ds(row0, nrows), :] = y * inv * g_ref[:, :]

        pf_cw = {0: pf_m1}
        pf_ccw = {0: pf_p1}
        idx4 = (pos + N_HOP) % N_RING
        pf4 = None

        DD = {"cw": {(0, 0): cw0[0], (0, 1): cw0[1]},
              "ccw": {(0, 0): ccw0[0], (0, 1): ccw0[1]}}
        HOP3_J = {"cw": 0, "ccw": 1}
        CRED = {"cw": (b_credit, (left_x, left_y, my_z)),
                "ccw": (c_credit, (right_x, right_y, my_z))}

        for h in range(N_HOP):
            for j in range(2):
                for dirn in ("cw", "ccw"):
                    if h == N_HOP - 1 and j != HOP3_J[dirn]:
                        continue
                    dd = DD[dirn][(h, j)]
                    dd.wait_recv()
                    dd.wait_send()
                    sem, up_dev = CRED[dirn]
                    if h == 1 or (h == 2 and j == HOP3_J[dirn]):
                        pl.semaphore_signal(
                            sem, inc=1, device_id=up_dev,
                            device_id_type=pl.DeviceIdType.MESH,
                        )
                    nxt = h + 1
                    if nxt < N_HOP and (
                        nxt < N_HOP - 1 or j == HOP3_J[dirn]
                    ):
                        if nxt >= 2:
                            pl.semaphore_wait(sem, 1)
                        nd = mk_fwd(dirn, nxt, j)
                        DD[dirn][(nxt, j)] = nd
                        nd.start()

            if h < N_HOP - 1:
                ln_from(pf_cw[h], ((pos - h - 1) % N_RING) * PIECE,
                        b_comm[h % 2])
                ln_from(pf_ccw[h], ((pos + h + 1) % N_RING) * PIECE,
                        c_comm[h % 2])
                if h == 0:
                    ln_from(pf_own, my_off, out_ref[pl.ds(my_off, PIECE), :])
                if h + 2 < N_HOP:
                    pf_cw[h + 1] = prefetch_piece((pos - h - 2) % N_RING)
                    pf_ccw[h + 1] = prefetch_piece((pos + h + 2) % N_RING)
                else:
                    pf4 = prefetch_piece(idx4)
            else:
                ln_from(pf4, idx4 * PIECE, b_comm[h % 2, :HALF, :],
                        nrows=HALF, sub_off=0)
                ln_from(pf4, idx4 * PIECE + HALF, c_comm[h % 2, HALF:, :],
                        nrows=HALF, sub_off=1, wait=False)

    return pl.pallas_call(
        body,
        out_shape=jax.ShapeDtypeStruct((M, D), jnp.float32),
        in_specs=[
            pl.BlockSpec(memory_space=pl.ANY),
            pl.BlockSpec(memory_space=pl.ANY),
            pl.BlockSpec(memory_space=pltpu.VMEM),
        ],
        out_specs=pl.BlockSpec(memory_space=pltpu.VMEM),
        scratch_shapes=[
            pltpu.VMEM((PIECE, D), jnp.float32),
            pltpu.VMEM((N_SUB, 2, QTR, D), jnp.float32),
            pltpu.VMEM((2, PIECE, D), jnp.float32),
            pltpu.VMEM((2, PIECE, D), jnp.float32),
            pltpu.VMEM((4, PIECE, D), jnp.float32),
            pltpu.SemaphoreType.DMA((N_SUB, 2)),
            pltpu.SemaphoreType.DMA((N_SUB, 2)),
            pltpu.SemaphoreType.DMA((2, 2)),
            pltpu.SemaphoreType.DMA((2, 2)),
            pltpu.SemaphoreType.DMA((2, 2)),
            pltpu.SemaphoreType.DMA((2, 2)),
            pltpu.SemaphoreType.REGULAR,
            pltpu.SemaphoreType.REGULAR,
            pltpu.SemaphoreType.DMA,
            pltpu.SemaphoreType.DMA((4,)),
        ],
        compiler_params=pltpu.CompilerParams(
            vmem_limit_bytes=100 * 1024 * 1024,
            collective_id=0,
        ),
    )(x, resid, g)


# device time: 146077 ns/iter; 1.0021x vs baseline; 1.0021x over previous
import jax
import jax.numpy as jnp
from jax import lax
from jax.experimental import pallas as pl
from jax.experimental.pallas import tpu as pltpu

N_Z = 4
N_RING = 8
M = 2048
D = 2048
PIECE = M // N_RING
HALF = PIECE // 2
N_SUB = 4
QTR = PIECE // N_SUB
N_HOP = 4


def _ring_coords(p):
    px = (p >= 4).astype(jnp.int32)
    py = jnp.where(p < 4, p, 7 - p)
    return px, py


def kernel(partial, resid, gamma):
    x = partial.reshape(M, D)
    g = gamma.reshape(1, D)

    def body(x_hbm, resid_hbm, g_ref, out_ref,
             acc_ref, a_comm, b_comm, c_comm, stage_ref,
             a_send, a_recv, b_send, b_recv, c_send, c_recv,
             b_credit, c_credit, local_sem, stage_sems):
        my_x = lax.axis_index("x")
        my_y = lax.axis_index("y")
        my_z = lax.axis_index("z")

        pos = jnp.where(my_x == 0, my_y, 7 - my_y)
        right_x, right_y = _ring_coords((pos + 1) % N_RING)
        left_x, left_y = _ring_coords((pos + 7) % N_RING)

        my_off = pos * PIECE

        is_edge = jnp.logical_or(my_z == 0, my_z == 3)
        partner = my_z ^ 1
        other_mid = 3 - my_z

        bar = pltpu.get_barrier_semaphore()
        for did in (
            (my_x, my_y, partner),
            (left_x, left_y, my_z),
            (right_x, right_y, my_z),
        ):
            pl.semaphore_signal(
                bar, inc=1, device_id=did,
                device_id_type=pl.DeviceIdType.MESH,
            )

        @pl.when(jnp.logical_not(is_edge))
        def _():
            pl.semaphore_signal(
                bar, inc=1, device_id=(my_x, my_y, other_mid),
                device_id_type=pl.DeviceIdType.MESH,
            )

        pl.semaphore_wait(bar, 3)

        @pl.when(jnp.logical_not(is_edge))
        def _():
            pl.semaphore_wait(bar, 1)

        cp = pltpu.make_async_copy(
            x_hbm.at[pl.ds(my_off, PIECE), :], acc_ref, local_sem
        )
        cp.start()

        pf_ctr = [0]

        def prefetch_piece(idx):
            slot = pf_ctr[0] % 4
            pf_ctr[0] += 1
            dma = pltpu.make_async_copy(
                resid_hbm.at[pl.ds(idx * PIECE, PIECE), :],
                stage_ref.at[slot],
                stage_sems.at[slot],
            )
            dma.start()
            return (dma, slot)

        pf_m1 = prefetch_piece((pos - 1) % N_RING)
        pf_p1 = prefetch_piece((pos + 1) % N_RING)
        pf_own = prefetch_piece(pos)

        cp.wait()

        def a_rdma(sub, slot, src, dst_z):
            return pltpu.make_async_remote_copy(
                src_ref=src,
                dst_ref=a_comm.at[sub, slot],
                send_sem=a_send.at[sub, slot],
                recv_sem=a_recv.at[sub, slot],
                device_id=(my_x, my_y, dst_z),
                device_id_type=pl.DeviceIdType.MESH,
            )

        def qtr(sub, ref=None, base=0):
            r = acc_ref if ref is None else ref
            return r.at[pl.ds(base + sub * QTR, QTR), :]

        def half(sub, ref=None, base=0):
            r = acc_ref if ref is None else ref
            return r.at[pl.ds(base + sub * HALF, HALF), :]

        def mk_fwd(dirn, h, j):
            comm, ss, rs = (
                (b_comm, b_send, b_recv) if dirn == "cw"
                else (c_comm, c_send, c_recv)
            )
            dev = (
                (right_x, right_y, my_z) if dirn == "cw"
                else (left_x, left_y, my_z)
            )
            if h == 0:
                src = half(j, out_ref, my_off)
            else:
                src = comm.at[(h - 1) % 2, pl.ds(j * HALF, HALF), :]
            return pltpu.make_async_remote_copy(
                src_ref=src,
                dst_ref=comm.at[h % 2, pl.ds(j * HALF, HALF), :],
                send_sem=ss.at[h % 2, j],
                recv_sem=rs.at[h % 2, j],
                device_id=dev,
                device_id_type=pl.DeviceIdType.MESH,
            )

        cw0 = [mk_fwd("cw", 0, j) for j in range(2)]
        ccw0 = [mk_fwd("ccw", 0, j) for j in range(2)]

        @pl.when(is_edge)
        def _():
            rd = [a_rdma(s, 0, qtr(s), partner) for s in range(N_SUB)]
            for s in range(N_SUB):
                rd[s].start()
            for s in range(N_SUB):
                fin = a_rdma(s, 1, qtr(s), partner)
                fin.wait_recv()
                out_ref[pl.ds(my_off + s * QTR, QTR), :] = a_comm[s, 1]
                if s % 2 == 1:
                    cw0[s // 2].start()
                    ccw0[s // 2].start()
            for s in range(N_SUB):
                rd[s].wait_send()

        @pl.when(jnp.logical_not(is_edge))
        def _():
            rcv = [a_rdma(s, 0, qtr(s), partner) for s in range(N_SUB)]
            ex = [None] * N_SUB
            for s in range(N_SUB):
                rcv[s].wait_recv()
                acc_ref[pl.ds(s * QTR, QTR), :] = (
                    acc_ref[pl.ds(s * QTR, QTR), :] + a_comm[s, 0]
                )
                ex[s] = a_rdma(s, 1, qtr(s), other_mid)
                ex[s].start()
            fin = [None] * N_SUB
            for s in range(N_SUB):
                ex[s].wait()
                out_ref[pl.ds(my_off + s * QTR, QTR), :] = (
                    acc_ref[pl.ds(s * QTR, QTR), :] + a_comm[s, 1]
                )
                fin[s] = a_rdma(
                    s, 1, qtr(s, out_ref, my_off), partner
                )
                fin[s].start()
                if s % 2 == 1:
                    cw0[s // 2].start()
                    ccw0[s // 2].start()
            for s in range(N_SUB):
                fin[s].wait_send()

        def ln_from(pf, row0, src_block, nrows=PIECE, sub_off=0, wait=True):
            dma, slot = pf
            if wait:
                dma.wait()
            if nrows == PIECE:
                st = stage_ref[slot]
            elif sub_off == 0:
                st = stage_ref[slot, :HALF, :]
            else:
                st = stage_ref[slot, HALF:, :]
            y = src_block + st
            ms = jnp.mean(y * y, axis=1, keepdims=True)
            inv = lax.rsqrt(ms + 1e-6)
            out_ref[pl.ds(row0, nrows), :] = y * inv * g_ref[:, :]

        pf_cw = {0: pf_m1}
        pf_ccw = {0: pf_p1}
        idx4 = (pos + N_HOP) % N_RING
        pf4 = None

        DD = {"cw": {(0, 0): cw0[0], (0, 1): cw0[1]},
              "ccw": {(0, 0): ccw0[0], (0, 1): ccw0[1]}}
        HOP3_J = {"cw": 0, "ccw": 1}
        CRED = {"cw": (b_credit, (left_x, left_y, my_z)),
                "ccw": (c_credit, (right_x, right_y, my_z))}

        for h in range(N_HOP):
            for j in range(2):
                for dirn in ("cw", "ccw"):
                    if h == N_HOP - 1 and j != HOP3_J[dirn]:
                        continue
                    dd = DD[dirn][(h, j)]
                    dd.wait_recv()
                    dd.wait_send()
                    sem, up_dev = CRED[dirn]
                    if h == 1 or (h == 2 and j == HOP3_J[dirn]):
                        pl.semaphore_signal(
                            sem, inc=1, device_id=up_dev,
                            device_id_type=pl.DeviceIdType.MESH,
                        )
                    nxt = h + 1
                    if nxt < N_HOP and (
                        nxt < N_HOP - 1 or j == HOP3_J[dirn]
                    ):
                        if nxt >= 2:
                            pl.semaphore_wait(sem, 1)
                        nd = mk_fwd(dirn, nxt, j)
                        DD[dirn][(nxt, j)] = nd
                        nd.start()

            if h < N_HOP - 1:
                ln_from(pf_cw[h], ((pos - h - 1) % N_RING) * PIECE,
                        b_comm[h % 2])
                ln_from(pf_ccw[h], ((pos + h + 1) % N_RING) * PIECE,
                        c_comm[h % 2])
                if h == 0:
                    ln_from(pf_own, my_off, out_ref[pl.ds(my_off, PIECE), :])
                if h + 2 < N_HOP:
                    pf_cw[h + 1] = prefetch_piece((pos - h - 2) % N_RING)
                    pf_ccw[h + 1] = prefetch_piece((pos + h + 2) % N_RING)
                else:
                    pf4 = prefetch_piece(idx4)
            else:
                ln_from(pf4, idx4 * PIECE, b_comm[h % 2, :HALF, :],
                        nrows=HALF, sub_off=0)
                ln_from(pf4, idx4 * PIECE + HALF, c_comm[h % 2, HALF:, :],
                        nrows=HALF, sub_off=1, wait=False)

    return pl.pallas_call(
        body,
        out_shape=jax.ShapeDtypeStruct((M, D), jnp.float32),
        in_specs=[
            pl.BlockSpec(memory_space=pl.ANY),
            pl.BlockSpec(memory_space=pl.ANY),
            pl.BlockSpec(memory_space=pltpu.VMEM),
        ],
        out_specs=pl.BlockSpec(memory_space=pltpu.VMEM),
        scratch_shapes=[
            pltpu.VMEM((PIECE, D), jnp.float32),
            pltpu.VMEM((N_SUB, 2, QTR, D), jnp.float32),
            pltpu.VMEM((2, PIECE, D), jnp.float32),
            pltpu.VMEM((2, PIECE, D), jnp.float32),
            pltpu.VMEM((4, PIECE, D), jnp.float32),
            pltpu.SemaphoreType.DMA((N_SUB, 2)),
            pltpu.SemaphoreType.DMA((N_SUB, 2)),
            pltpu.SemaphoreType.DMA((2, 2)),
            pltpu.SemaphoreType.DMA((2, 2)),
            pltpu.SemaphoreType.DMA((2, 2)),
            pltpu.SemaphoreType.DMA((2, 2)),
            pltpu.SemaphoreType.REGULAR,
            pltpu.SemaphoreType.REGULAR,
            pltpu.SemaphoreType.DMA,
            pltpu.SemaphoreType.DMA((4,)),
        ],
        compiler_params=pltpu.CompilerParams(
            vmem_limit_bytes=100 * 1024 * 1024,
            collective_id=0,
        ),
    )(x, resid, g)
